# baseline (speedup 1.0000x reference)
import jax
import jax.numpy as jnp
from jax import lax
from jax.experimental import pallas as pl
from jax.experimental.pallas import tpu as pltpu
from jax.experimental.pallas import tpu_sc as plsc

_N_ORB = 19
_MAX_E = 15
_D = 128
_N_ELEM = 96
_N_ATOMS = 10000

_NC = 2
_NS = 16
_NW = _NC * _NS

_C = 16
_BPW = 312
_NCH = 20
_EPW = _N_ELEM // _NS
_CI_PAD = 24


def _sc_body(z_hbm, ci_hbm, tabs_hbm, out_hbm, comb_hbm,
             idx_v, rows_a, zb0, rb0, zb1, rb1, sem_a, sem0, sem1):
    c = lax.axis_index("c")
    s = lax.axis_index("s")
    wid = s * _NC + c

    for j in range(_EPW):
        e = s * _EPW + j
        pltpu.sync_copy(ci_hbm.at[e], idx_v)
        pltpu.async_copy(tabs_hbm.at[idx_v], rows_a, sem_a).wait()
        pltpu.sync_copy(rows_a.at[pl.ds(0, _N_ORB)], comb_hbm.at[e])
    plsc.subcore_barrier()

    base = wid * _BPW
    wend = base + _BPW + jnp.where(wid == _NW - 1, _N_ATOMS - _NW * _BPW, 0)
    slots = ((zb0, rb0, sem0), (zb1, rb1, sem1))

    def cb(k):
        return jnp.minimum(base + k * _C, wend - _C)

    def start(k):
        zb, rb, sem = slots[k % 2]
        pltpu.sync_copy(z_hbm.at[pl.ds(cb(k), _C)], zb)
        pltpu.async_copy(comb_hbm.at[zb], rb, sem)

    def finish(k):
        zb, rb, sem = slots[k % 2]
        pltpu.make_async_copy(comb_hbm.at[zb], rb, sem).wait()
        pltpu.sync_copy(rb, out_hbm.at[pl.ds(cb(k), _C)])

    start(0)
    for k in range(_NCH):
        if k + 1 < _NCH:
            start(k + 1)
        else:
            @pl.when(wid == _NW - 1)
            def _():
                start(_NCH)
        finish(k)

    @pl.when(wid == _NW - 1)
    def _():
        finish(_NCH)


def kernel(z, elec_table, tables):
    z = z.astype(jnp.int32)
    tabs = tables.at[:, 0, :].set(0.0).reshape(_N_ORB * _MAX_E, _D)
    ci = elec_table.astype(jnp.int32) + (jnp.arange(_N_ORB, dtype=jnp.int32) * _MAX_E)[None, :]
    ci = jnp.pad(ci, ((0, 0), (0, _CI_PAD - _N_ORB)))

    mesh = plsc.VectorSubcoreMesh(core_axis_name="c", subcore_axis_name="s")
    out = pl.kernel(
        _sc_body,
        out_type=jax.ShapeDtypeStruct((_N_ATOMS, _N_ORB, _D), jnp.float32),
        mesh=mesh,
        scratch_types=[
            pltpu.HBM((_N_ELEM, _N_ORB, _D), jnp.float32),
            pltpu.VMEM((_CI_PAD,), jnp.int32),
            pltpu.VMEM((_CI_PAD, _D), jnp.float32),
            pltpu.VMEM((_C,), jnp.int32),
            pltpu.VMEM((_C, _N_ORB, _D), jnp.float32),
            pltpu.VMEM((_C,), jnp.int32),
            pltpu.VMEM((_C, _N_ORB, _D), jnp.float32),
            pltpu.SemaphoreType.DMA,
            pltpu.SemaphoreType.DMA,
            pltpu.SemaphoreType.DMA,
        ],
    )(z, ci, tabs)
    return out


# speedup vs baseline: 1.0002x; 1.0002x over previous
import jax
import jax.numpy as jnp
from jax import lax
from jax.experimental import pallas as pl
from jax.experimental.pallas import tpu as pltpu
from jax.experimental.pallas import tpu_sc as plsc

_N_ORB = 19
_MAX_E = 15
_D = 128
_N_ELEM = 96
_N_ATOMS = 10000

_NC = 2
_NS = 16
_NW = _NC * _NS

_C = 16
_BPW = 312
_NCH = 20
_EPW = _N_ELEM // _NS
_CI_PAD = 24


def _sc_body(z_hbm, ci_hbm, tabs_hbm, out_hbm, comb_hbm,
             idx_v, rows_a, zb0, rb0, zb1, rb1, sem_a, sem0, sem1):
    c = lax.axis_index("c")
    s = lax.axis_index("s")
    wid = s * _NC + c

    for j in range(_EPW):
        e = s * _EPW + j
        pltpu.sync_copy(ci_hbm.at[e], idx_v)
        pltpu.async_copy(tabs_hbm.at[idx_v], rows_a, sem_a).wait()
        pltpu.sync_copy(rows_a.at[pl.ds(0, _N_ORB)], comb_hbm.at[e])
    plsc.subcore_barrier()

    base = wid * _BPW
    wend = base + _BPW + jnp.where(wid == _NW - 1, _N_ATOMS - _NW * _BPW, 0)
    slots = ((zb0, rb0, sem0), (zb1, rb1, sem1))

    def cb(k):
        return jnp.minimum(base + k * _C, wend - _C)

    def start(k):
        zb, rb, sem = slots[k % 2]
        pltpu.sync_copy(z_hbm.at[pl.ds(cb(k), _C)], zb)
        pltpu.async_copy(comb_hbm.at[zb], rb, sem)

    def finish(k):
        zb, rb, sem = slots[k % 2]
        pltpu.make_async_copy(comb_hbm.at[zb], rb, sem).wait()
        pltpu.sync_copy(rb, out_hbm.at[pl.ds(cb(k), _C)])

    start(0)
    for k in range(_NCH):
        if k + 1 < _NCH:
            start(k + 1)
        else:
            @pl.when(wid == _NW - 1)
            def _():
                start(_NCH)
        finish(k)

    @pl.when(wid == _NW - 1)
    def _():
        finish(_NCH)


def kernel(z, elec_table, tables):
    z = z.astype(jnp.int32)
    tabs = tables.at[:, 0, :].set(0.0).reshape(_N_ORB * _MAX_E, _D)
    ci = elec_table.astype(jnp.int32) + (jnp.arange(_N_ORB, dtype=jnp.int32) * _MAX_E)[None, :]
    ci = jnp.pad(ci, ((0, 0), (0, _CI_PAD - _N_ORB)))

    mesh = plsc.VectorSubcoreMesh(core_axis_name="c", subcore_axis_name="s")
    out = pl.kernel(
        _sc_body,
        out_type=jax.ShapeDtypeStruct((_N_ATOMS, _N_ORB, _D), jnp.float32),
        mesh=mesh,
        compiler_params=pltpu.CompilerParams(use_tc_tiling_on_sc=True),
        scratch_types=[
            pltpu.HBM((_N_ELEM, _N_ORB, _D), jnp.float32),
            pltpu.VMEM((_CI_PAD,), jnp.int32),
            pltpu.VMEM((_CI_PAD, _D), jnp.float32),
            pltpu.VMEM((_C,), jnp.int32),
            pltpu.VMEM((_C, _N_ORB, _D), jnp.float32),
            pltpu.VMEM((_C,), jnp.int32),
            pltpu.VMEM((_C, _N_ORB, _D), jnp.float32),
            pltpu.SemaphoreType.DMA,
            pltpu.SemaphoreType.DMA,
            pltpu.SemaphoreType.DMA,
        ],
    )(z, ci, tabs)
    return out


# speedup vs baseline: 1.4724x; 1.4720x over previous
import jax
import jax.numpy as jnp
from jax import lax
from jax.experimental import pallas as pl
from jax.experimental.pallas import tpu as pltpu
from jax.experimental.pallas import tpu_sc as plsc

_N_ORB = 19
_MAX_E = 15
_D = 128
_N_ELEM = 96
_N_ATOMS = 10000

_NC = 2
_NS = 16
_NW = _NC * _NS

_C = 104
_BPW = 312
_NCH = 3


def _sc_body(z_hbm, ci_hbm, tabs_hbm, out_hbm, comb_hbm,
             ci_v, ta_v, zb_v, rb0, rb1, sem_a, sem0, sem1):
    c = lax.axis_index("c")
    s = lax.axis_index("s")
    wid = s * _NC + c

    for r in range(2):
        i = s + _NS * r

        @pl.when(i < _N_ORB)
        def _():
            pltpu.sync_copy(ci_hbm.at[i], ci_v)
            pltpu.async_copy(tabs_hbm.at[ci_v], ta_v, sem_a).wait()
            pltpu.sync_copy(ta_v, comb_hbm.at[i])

    plsc.subcore_barrier()

    base = wid * _BPW
    wend = base + _BPW + jnp.where(wid == _NW - 1, _N_ATOMS - _NW * _BPW, 0)
    nch = _NCH + jnp.where(wid == _NW - 1, 1, 0)
    slots = ((rb0, sem0), (rb1, sem1))

    def chunk(k, carry):
        b = jnp.minimum(base + k * _C, wend - _C)
        pltpu.sync_copy(z_hbm.at[pl.ds(b, _C)], zb_v)

        def start(i):
            rb, sem = slots[i % 2]
            pltpu.async_copy(comb_hbm.at[i].at[zb_v], rb, sem)

        def finish(i):
            rb, sem = slots[i % 2]
            pltpu.make_async_copy(comb_hbm.at[i].at[zb_v], rb, sem).wait()
            pltpu.sync_copy(rb, out_hbm.at[i, pl.ds(b, _C)])

        start(0)
        for i in range(_N_ORB):
            if i + 1 < _N_ORB:
                start(i + 1)
            finish(i)
        return carry

    lax.fori_loop(0, nch, chunk, None)


def kernel(z, elec_table, tables):
    z = z.astype(jnp.int32)
    tabs = tables.at[:, 0, :].set(0.0).reshape(_N_ORB * _MAX_E, _D)
    ci_t = (elec_table.astype(jnp.int32)
            + (jnp.arange(_N_ORB, dtype=jnp.int32) * _MAX_E)[None, :]).T

    mesh = plsc.VectorSubcoreMesh(core_axis_name="c", subcore_axis_name="s")
    out_t = pl.kernel(
        _sc_body,
        out_type=jax.ShapeDtypeStruct((_N_ORB, _N_ATOMS, _D), jnp.float32),
        mesh=mesh,
        scratch_types=[
            pltpu.HBM((_N_ORB, _N_ELEM, _D), jnp.float32),
            pltpu.VMEM((_N_ELEM,), jnp.int32),
            pltpu.VMEM((_N_ELEM, _D), jnp.float32),
            pltpu.VMEM((_C,), jnp.int32),
            pltpu.VMEM((_C, _D), jnp.float32),
            pltpu.VMEM((_C, _D), jnp.float32),
            pltpu.SemaphoreType.DMA,
            pltpu.SemaphoreType.DMA,
            pltpu.SemaphoreType.DMA,
        ],
    )(z, ci_t, tabs)
    return out_t.transpose(1, 0, 2)


# speedup vs baseline: 2.9494x; 2.0032x over previous
import jax
import jax.numpy as jnp
from jax import lax
from jax.experimental import pallas as pl
from jax.experimental.pallas import tpu as pltpu
from jax.experimental.pallas import tpu_sc as plsc

_N_ORB = 19
_MAX_E = 15
_D = 128
_N_ELEM = 96
_N_ATOMS = 10000

_NC = 2
_NS = 16
_NW = _NC * _NS

_C = 104
_BPW = 312
_NCH = 3


def _sc_body(z_hbm, ci_hbm, tabs_hbm, out_hbm, comb_hbm,
             ci_v, ta_v, zb_v, rb0, rb1, sem_a, sem0, sem1):
    c = lax.axis_index("c")
    s = lax.axis_index("s")
    wid = s * _NC + c

    for r in range(2):
        i = s + _NS * r

        @pl.when(i < _N_ORB)
        def _():
            pltpu.sync_copy(ci_hbm.at[i], ci_v)
            pltpu.async_copy(tabs_hbm.at[ci_v], ta_v, sem_a).wait()
            pltpu.sync_copy(ta_v, comb_hbm.at[i])

    plsc.subcore_barrier()

    base = wid * _BPW
    wend = base + _BPW + jnp.where(wid == _NW - 1, _N_ATOMS - _NW * _BPW, 0)
    nch = _NCH + jnp.where(wid == _NW - 1, 1, 0)
    slots = ((rb0, sem0), (rb1, sem1))

    def chunk(k, carry):
        b = jnp.minimum(base + k * _C, wend - _C)
        pltpu.sync_copy(z_hbm.at[pl.ds(b, _C)], zb_v)

        def start(i):
            rb, sem = slots[i % 2]
            pltpu.async_copy(comb_hbm.at[i].at[zb_v], rb, sem)

        def finish(i):
            rb, sem = slots[i % 2]
            pltpu.make_async_copy(comb_hbm.at[i].at[zb_v], rb, sem).wait()
            pltpu.sync_copy(rb, out_hbm.at[i, pl.ds(b, _C)])

        start(0)
        for i in range(_N_ORB):
            if i + 1 < _N_ORB:
                start(i + 1)
            finish(i)
        return carry

    lax.fori_loop(0, nch, chunk, None)


def kernel(z, elec_table, tables):
    z = z.astype(jnp.int32)
    tabs = tables.at[:, 0, :].set(0.0).reshape(_N_ORB * _MAX_E, _D)
    ci_t = (elec_table.astype(jnp.int32)
            + (jnp.arange(_N_ORB, dtype=jnp.int32) * _MAX_E)[None, :]).T

    mesh = plsc.VectorSubcoreMesh(core_axis_name="c", subcore_axis_name="s")
    out_t = pl.kernel(
        _sc_body,
        out_type=jax.ShapeDtypeStruct((_N_ORB, _N_ATOMS, _D), jnp.float32),
        mesh=mesh,
        scratch_types=[
            pltpu.VMEM_SHARED((_N_ORB, _N_ELEM, _D), jnp.float32),
            pltpu.VMEM((_N_ELEM,), jnp.int32),
            pltpu.VMEM((_N_ELEM, _D), jnp.float32),
            pltpu.VMEM((_C,), jnp.int32),
            pltpu.VMEM((_C, _D), jnp.float32),
            pltpu.VMEM((_C, _D), jnp.float32),
            pltpu.SemaphoreType.DMA,
            pltpu.SemaphoreType.DMA,
            pltpu.SemaphoreType.DMA,
        ],
    )(z, ci_t, tabs)
    return out_t.transpose(1, 0, 2)


# speedup vs baseline: 2.9977x; 1.0164x over previous
import jax
import jax.numpy as jnp
from jax import lax
from jax.experimental import pallas as pl
from jax.experimental.pallas import tpu as pltpu
from jax.experimental.pallas import tpu_sc as plsc

_N_ORB = 19
_MAX_E = 15
_D = 128
_N_ELEM = 96
_N_ATOMS = 10000

_NC = 2
_NS = 16
_NW = _NC * _NS

_C = 104
_BPW = 312
_NCH = 3


def _sc_body(z_hbm, ci_hbm, tabs_hbm, out_hbm, comb_hbm,
             ci_v, ta_v, zb0_v, zb1_v, rb0, rb1, rb2,
             sem_a, gs0, gs1, gs2, ts0, ts1, ts2):
    c = lax.axis_index("c")
    s = lax.axis_index("s")
    wid = s * _NC + c

    for r in range(2):
        i = s + _NS * r

        @pl.when(i < _N_ORB)
        def _():
            pltpu.sync_copy(ci_hbm.at[i], ci_v)
            pltpu.async_copy(tabs_hbm.at[ci_v], ta_v, sem_a).wait()
            pltpu.sync_copy(ta_v, comb_hbm.at[i])

    plsc.subcore_barrier()

    base = wid * _BPW
    wend = base + _BPW + jnp.where(wid == _NW - 1, _N_ATOMS - _NW * _BPW, 0)
    zbs = (zb0_v, zb1_v)
    rbs = (rb0, rb1, rb2)
    gsem = (gs0, gs1, gs2)
    tsem = (ts0, ts1, ts2)
    n_t = _NCH * _N_ORB

    def cbase(k):
        return jnp.minimum(base + k * _C, wend - _C)

    def gather_start(t):
        k, i = divmod(t, _N_ORB)
        pltpu.async_copy(comb_hbm.at[i].at[zbs[k % 2]], rbs[t % 3], gsem[t % 3])

    def gather_wait(t):
        k, i = divmod(t, _N_ORB)
        pltpu.make_async_copy(
            comb_hbm.at[i].at[zbs[k % 2]], rbs[t % 3], gsem[t % 3]).wait()

    def store_start(t):
        k, i = divmod(t, _N_ORB)
        pltpu.async_copy(rbs[t % 3], out_hbm.at[i, pl.ds(cbase(k), _C)], tsem[t % 3])

    def store_wait(t):
        k, i = divmod(t, _N_ORB)
        pltpu.make_async_copy(
            rbs[t % 3], out_hbm.at[i, pl.ds(cbase(k), _C)], tsem[t % 3]).wait()

    def stage(t):
        tn = t + 1

        def issue():
            if tn % _N_ORB == 0:
                pltpu.sync_copy(
                    z_hbm.at[pl.ds(cbase(tn // _N_ORB), _C)], zbs[(tn // _N_ORB) % 2])
            if tn >= 3:
                store_wait(tn - 3)
            gather_start(tn)

        if tn < n_t:
            issue()
        elif tn < n_t + _N_ORB:
            pl.when(wid == _NW - 1)(issue)
        gather_wait(t)
        store_start(t)

    pltpu.sync_copy(z_hbm.at[pl.ds(cbase(0), _C)], zbs[0])
    gather_start(0)
    for t in range(n_t):
        stage(t)
    for t in range(n_t, n_t + _N_ORB):
        @pl.when(wid == _NW - 1)
        def _():
            stage(t)
    for slot in range(3):
        pltpu.make_async_copy(
            rbs[slot], out_hbm.at[0, pl.ds(base, _C)], tsem[slot]).wait()


def kernel(z, elec_table, tables):
    z = z.astype(jnp.int32)
    tabs = tables.at[:, 0, :].set(0.0).reshape(_N_ORB * _MAX_E, _D)
    ci_t = (elec_table.astype(jnp.int32)
            + (jnp.arange(_N_ORB, dtype=jnp.int32) * _MAX_E)[None, :]).T

    mesh = plsc.VectorSubcoreMesh(core_axis_name="c", subcore_axis_name="s")
    out_t = pl.kernel(
        _sc_body,
        out_type=jax.ShapeDtypeStruct((_N_ORB, _N_ATOMS, _D), jnp.float32),
        mesh=mesh,
        scratch_types=[
            pltpu.VMEM_SHARED((_N_ORB, _N_ELEM, _D), jnp.float32),
            pltpu.VMEM((_N_ELEM,), jnp.int32),
            pltpu.VMEM((_N_ELEM, _D), jnp.float32),
            pltpu.VMEM((_C,), jnp.int32),
            pltpu.VMEM((_C,), jnp.int32),
            pltpu.VMEM((_C, _D), jnp.float32),
            pltpu.VMEM((_C, _D), jnp.float32),
            pltpu.VMEM((_C, _D), jnp.float32),
            pltpu.SemaphoreType.DMA,
            pltpu.SemaphoreType.DMA,
            pltpu.SemaphoreType.DMA,
            pltpu.SemaphoreType.DMA,
            pltpu.SemaphoreType.DMA,
            pltpu.SemaphoreType.DMA,
            pltpu.SemaphoreType.DMA,
        ],
    )(z, ci_t, tabs)
    return out_t.transpose(1, 0, 2)


# speedup vs baseline: 3.1163x; 1.0395x over previous
import jax
import jax.numpy as jnp
from jax import lax
from jax.experimental import pallas as pl
from jax.experimental.pallas import tpu as pltpu
from jax.experimental.pallas import tpu_sc as plsc

_N_ORB = 19
_MAX_E = 15
_D = 128
_N_ELEM = 96
_N_ATOMS = 10000

_NC = 2
_NS = 16
_NW = _NC * _NS

_C = 104
_BPW = 312
_NCH = 3
_CT = 8


def _sc_body(z_hbm, ci_hbm, tabs_hbm, out_hbm, comb_hbm,
             ci_v, ta_v, zb0_v, zb1_v, rb0, rb1, rb2, zt_v, rt0, rt1,
             sem_a, gs0, gs1, gs2, ts0, ts1, ts2):
    c = lax.axis_index("c")
    s = lax.axis_index("s")
    wid = s * _NC + c

    for r in range(2):
        i = s + _NS * r

        @pl.when(i < _N_ORB)
        def _():
            pltpu.sync_copy(ci_hbm.at[i], ci_v)
            pltpu.async_copy(tabs_hbm.at[ci_v], ta_v, sem_a).wait()
            pltpu.sync_copy(ta_v, comb_hbm.at[i])

    plsc.subcore_barrier()

    base = wid * _BPW
    zbs = (zb0_v, zb1_v)
    rbs = (rb0, rb1, rb2)
    gsem = (gs0, gs1, gs2)
    tsem = (ts0, ts1, ts2)
    n_t = _NCH * _N_ORB

    def cbase(k):
        return base + k * _C

    def gather_start(t):
        k, i = divmod(t, _N_ORB)
        pltpu.async_copy(comb_hbm.at[i].at[zbs[k % 2]], rbs[t % 3], gsem[t % 3])

    def gather_wait(t):
        k, i = divmod(t, _N_ORB)
        pltpu.make_async_copy(
            comb_hbm.at[i].at[zbs[k % 2]], rbs[t % 3], gsem[t % 3]).wait()

    def store_start(t):
        k, i = divmod(t, _N_ORB)
        pltpu.async_copy(rbs[t % 3], out_hbm.at[i, pl.ds(cbase(k), _C)], tsem[t % 3])

    def store_wait(t):
        k, i = divmod(t, _N_ORB)
        pltpu.make_async_copy(
            rbs[t % 3], out_hbm.at[i, pl.ds(cbase(k), _C)], tsem[t % 3]).wait()

    def stage(t):
        tn = t + 1
        if tn < n_t:
            if tn % _N_ORB == 0:
                pltpu.sync_copy(
                    z_hbm.at[pl.ds(cbase(tn // _N_ORB), _C)], zbs[(tn // _N_ORB) % 2])
            if tn >= 3:
                store_wait(tn - 3)
            gather_start(tn)
        gather_wait(t)
        store_start(t)

    pltpu.sync_copy(z_hbm.at[pl.ds(cbase(0), _C)], zbs[0])
    gather_start(0)
    for t in range(n_t):
        stage(t)
    for slot in range(3):
        pltpu.make_async_copy(
            rbs[slot], out_hbm.at[0, pl.ds(base, _C)], tsem[slot]).wait()

    @pl.when(wid >= _NW - 2)
    def _():
        tb = _NW * _BPW + _CT * (wid - (_NW - 2))
        pltpu.sync_copy(z_hbm.at[pl.ds(tb, _CT)], zt_v)
        trbs = (rt0, rt1)

        def tg_start(i):
            pltpu.async_copy(comb_hbm.at[i].at[zt_v], trbs[i % 2], gsem[i % 2])

        def tg_wait(i):
            pltpu.make_async_copy(
                comb_hbm.at[i].at[zt_v], trbs[i % 2], gsem[i % 2]).wait()

        def tst_start(i):
            pltpu.async_copy(trbs[i % 2], out_hbm.at[i, pl.ds(tb, _CT)], tsem[i % 2])

        def tst_wait(i):
            pltpu.make_async_copy(
                trbs[i % 2], out_hbm.at[i, pl.ds(tb, _CT)], tsem[i % 2]).wait()

        tg_start(0)
        for i in range(_N_ORB):
            if i + 1 < _N_ORB:
                if i >= 1:
                    tst_wait(i - 1)
                tg_start(i + 1)
            tg_wait(i)
            tst_start(i)
        tst_wait(_N_ORB - 2)
        tst_wait(_N_ORB - 1)


def kernel(z, elec_table, tables):
    z = z.astype(jnp.int32)
    tabs = tables.at[:, 0, :].set(0.0).reshape(_N_ORB * _MAX_E, _D)
    ci_t = (elec_table.astype(jnp.int32)
            + (jnp.arange(_N_ORB, dtype=jnp.int32) * _MAX_E)[None, :]).T

    mesh = plsc.VectorSubcoreMesh(core_axis_name="c", subcore_axis_name="s")
    out_t = pl.kernel(
        _sc_body,
        out_type=jax.ShapeDtypeStruct((_N_ORB, _N_ATOMS, _D), jnp.float32),
        mesh=mesh,
        scratch_types=[
            pltpu.VMEM_SHARED((_N_ORB, _N_ELEM, _D), jnp.float32),
            pltpu.VMEM((_N_ELEM,), jnp.int32),
            pltpu.VMEM((_N_ELEM, _D), jnp.float32),
            pltpu.VMEM((_C,), jnp.int32),
            pltpu.VMEM((_C,), jnp.int32),
            pltpu.VMEM((_C, _D), jnp.float32),
            pltpu.VMEM((_C, _D), jnp.float32),
            pltpu.VMEM((_C, _D), jnp.float32),
            pltpu.VMEM((_CT,), jnp.int32),
            pltpu.VMEM((_CT, _D), jnp.float32),
            pltpu.VMEM((_CT, _D), jnp.float32),
            pltpu.SemaphoreType.DMA,
            pltpu.SemaphoreType.DMA,
            pltpu.SemaphoreType.DMA,
            pltpu.SemaphoreType.DMA,
            pltpu.SemaphoreType.DMA,
            pltpu.SemaphoreType.DMA,
            pltpu.SemaphoreType.DMA,
        ],
    )(z, ci_t, tabs)
    return out_t.transpose(1, 0, 2)


# speedup vs baseline: 3.2988x; 1.0586x over previous
import jax
import jax.numpy as jnp
from jax import lax
from jax.experimental import pallas as pl
from jax.experimental.pallas import tpu as pltpu
from jax.experimental.pallas import tpu_sc as plsc

_N_ORB = 19
_MAX_E = 15
_D = 128
_N_ELEM = 96
_N_ATOMS = 10000

_NC = 2
_NS = 16
_NW = _NC * _NS

_C = 104
_BPW = 312
_NCH = 3
_CT = 8


def _sc_body(z_hbm, ci_hbm, tabs_hbm, out_hbm, comb_hbm,
             ci_v, ci2_v, ta_v, ta2_v, zb0_v, zb1_v, rb0, rb1, rb2,
             zt_v, rt0, rt1, sem_a, sem_a2, gs0, gs1, gs2, ts0, ts1, ts2):
    c = lax.axis_index("c")
    s = lax.axis_index("s")
    wid = s * _NC + c

    for r in range(2):
        i = s + _NS * r

        @pl.when(i < _N_ORB)
        def _():
            pltpu.sync_copy(ci_hbm.at[i], (ci_v, ci2_v)[r])
            pltpu.async_copy(tabs_hbm.at[(ci_v, ci2_v)[r]], (ta_v, ta2_v)[r],
                             (sem_a, sem_a2)[r])

    for r in range(2):
        i = s + _NS * r

        @pl.when(i < _N_ORB)
        def _():
            pltpu.make_async_copy(tabs_hbm.at[(ci_v, ci2_v)[r]], (ta_v, ta2_v)[r],
                                  (sem_a, sem_a2)[r]).wait()
            pltpu.sync_copy((ta_v, ta2_v)[r], comb_hbm.at[i])

    plsc.subcore_barrier()

    base = wid * _BPW
    zbs = (zb0_v, zb1_v)
    rbs = (rb0, rb1, rb2)
    gsem = (gs0, gs1, gs2)
    tsem = (ts0, ts1, ts2)
    n_t = _NCH * _N_ORB

    def cbase(k):
        return base + k * _C

    def gather_start(t):
        k, i = divmod(t, _N_ORB)
        pltpu.async_copy(comb_hbm.at[i].at[zbs[k % 2]], rbs[t % 3], gsem[t % 3])

    def gather_wait(t):
        k, i = divmod(t, _N_ORB)
        pltpu.make_async_copy(
            comb_hbm.at[i].at[zbs[k % 2]], rbs[t % 3], gsem[t % 3]).wait()

    def store_start(t):
        k, i = divmod(t, _N_ORB)
        pltpu.async_copy(rbs[t % 3], out_hbm.at[i, pl.ds(cbase(k), _C)], tsem[t % 3])

    def store_wait(t):
        k, i = divmod(t, _N_ORB)
        pltpu.make_async_copy(
            rbs[t % 3], out_hbm.at[i, pl.ds(cbase(k), _C)], tsem[t % 3]).wait()

    def stage(t):
        tn = t + 1
        if tn < n_t:
            if tn % _N_ORB == 0:
                pltpu.sync_copy(
                    z_hbm.at[pl.ds(cbase(tn // _N_ORB), _C)], zbs[(tn // _N_ORB) % 2])
            if tn >= 3:
                store_wait(tn - 3)
            gather_start(tn)
        gather_wait(t)
        store_start(t)

    pltpu.sync_copy(z_hbm.at[pl.ds(cbase(0), _C)], zbs[0])
    gather_start(0)
    for t in range(n_t):
        stage(t)
    for slot in range(3):
        pltpu.make_async_copy(
            rbs[slot], out_hbm.at[0, pl.ds(base, _C)], tsem[slot]).wait()

    @pl.when(wid >= _NW - 2)
    def _():
        tb = _NW * _BPW + _CT * (wid - (_NW - 2))
        pltpu.sync_copy(z_hbm.at[pl.ds(tb, _CT)], zt_v)
        trbs = (rt0, rt1)

        def tg_start(i):
            pltpu.async_copy(comb_hbm.at[i].at[zt_v], trbs[i % 2], gsem[i % 2])

        def tg_wait(i):
            pltpu.make_async_copy(
                comb_hbm.at[i].at[zt_v], trbs[i % 2], gsem[i % 2]).wait()

        def tst_start(i):
            pltpu.async_copy(trbs[i % 2], out_hbm.at[i, pl.ds(tb, _CT)], tsem[i % 2])

        def tst_wait(i):
            pltpu.make_async_copy(
                trbs[i % 2], out_hbm.at[i, pl.ds(tb, _CT)], tsem[i % 2]).wait()

        tg_start(0)
        for i in range(_N_ORB):
            if i + 1 < _N_ORB:
                if i >= 1:
                    tst_wait(i - 1)
                tg_start(i + 1)
            tg_wait(i)
            tst_start(i)
        tst_wait(_N_ORB - 2)
        tst_wait(_N_ORB - 1)


def kernel(z, elec_table, tables):
    z = z.astype(jnp.int32)
    tabs = tables.at[:, 0, :].set(0.0).reshape(_N_ORB * _MAX_E, _D)
    ci_t = (elec_table.astype(jnp.int32)
            + (jnp.arange(_N_ORB, dtype=jnp.int32) * _MAX_E)[None, :]).T

    mesh = plsc.VectorSubcoreMesh(core_axis_name="c", subcore_axis_name="s")
    out_t = pl.kernel(
        _sc_body,
        out_type=jax.ShapeDtypeStruct((_N_ORB, _N_ATOMS, _D), jnp.float32),
        mesh=mesh,
        scratch_types=[
            pltpu.VMEM_SHARED((_N_ORB, _N_ELEM, _D), jnp.float32),
            pltpu.VMEM((_N_ELEM,), jnp.int32),
            pltpu.VMEM((_N_ELEM,), jnp.int32),
            pltpu.VMEM((_N_ELEM, _D), jnp.float32),
            pltpu.VMEM((_N_ELEM, _D), jnp.float32),
            pltpu.VMEM((_C,), jnp.int32),
            pltpu.VMEM((_C,), jnp.int32),
            pltpu.VMEM((_C, _D), jnp.float32),
            pltpu.VMEM((_C, _D), jnp.float32),
            pltpu.VMEM((_C, _D), jnp.float32),
            pltpu.VMEM((_CT,), jnp.int32),
            pltpu.VMEM((_CT, _D), jnp.float32),
            pltpu.VMEM((_CT, _D), jnp.float32),
            pltpu.SemaphoreType.DMA,
            pltpu.SemaphoreType.DMA,
            pltpu.SemaphoreType.DMA,
            pltpu.SemaphoreType.DMA,
            pltpu.SemaphoreType.DMA,
            pltpu.SemaphoreType.DMA,
            pltpu.SemaphoreType.DMA,
            pltpu.SemaphoreType.DMA,
        ],
    )(z, ci_t, tabs)
    return out_t.transpose(1, 0, 2)
